# Initial kernel scaffold; baseline (speedup 1.0000x reference)
import jax
import jax.numpy as jnp
from jax.experimental import pallas as pl


def kernel(type_emb, type_ids, hex1, hex2):
    raise NotImplementedError("write your pallas kernel here")


# speedup vs baseline: 1.8586x; 1.8586x over previous
import functools

import jax
import jax.numpy as jnp
from jax import lax
from jax.experimental import pallas as pl
from jax.experimental.pallas import tpu as pltpu
from jax.experimental.pallas import tpu_sc as plsc

WIDTH_FULL = 17
WIDTH_PLAYABLE = 15
HEIGHT = 11
TYPE_EMB_DIM = 8
K = 16384
OUT_D = 14
LANES = 16
NC, NS = 2, 16
NW = NC * NS
IPW = K // NW
GROUPS = IPW // LANES
N_EMB_WORDS = 4 * TYPE_EMB_DIM


def _sc_body(emb_hbm, tid_hbm, h1_hbm, h2_hbm, out_hbm,
             emb_v, tid_v, h1_v, h2_v, out_v):
    wid = lax.axis_index("s") * NC + lax.axis_index("c")
    base = wid * IPW

    pltpu.sync_copy(emb_hbm, emb_v)
    pltpu.sync_copy(tid_hbm.at[pl.ds(base, IPW)], tid_v)
    pltpu.sync_copy(h1_hbm.at[pl.ds(base, IPW)], h1_v)
    pltpu.sync_copy(h2_hbm.at[pl.ds(base, IPW)], h2_v)

    lane = lax.iota(jnp.int32, LANES)

    def group(g, carry):
        off = g * LANES
        tid = tid_v[pl.ds(off, LANES)]
        h1 = h1_v[pl.ds(off, LANES)]
        h2 = h2_v[pl.ds(off, LANES)]
        row = (off + lane) * OUT_D

        tid8 = tid * TYPE_EMB_DIM
        for j in range(TYPE_EMB_DIM):
            ej = plsc.load_gather(emb_v, [tid8 + j])
            plsc.store_scatter(out_v, [row + j], ej)

        for h, jb in ((h1, TYPE_EMB_DIM), (h2, TYPE_EMB_DIM + 3)):
            y = lax.div(h, WIDTH_FULL)
            x = h - y * WIDTH_FULL
            valid = h >= 0
            xf = jnp.minimum(x, WIDTH_PLAYABLE - 1).astype(jnp.float32)
            yf = jnp.minimum(y, HEIGHT - 1).astype(jnp.float32)
            xf = jnp.where(valid, xf * (1.0 / (WIDTH_PLAYABLE - 1)), 0.0)
            yf = jnp.where(valid, yf * (1.0 / (HEIGHT - 1)), 0.0)
            one = jnp.where(valid, 1.0, 0.0)
            plsc.store_scatter(out_v, [row + jb], xf)
            plsc.store_scatter(out_v, [row + jb + 1], yf)
            plsc.store_scatter(out_v, [row + jb + 2], one)
        return carry

    lax.fori_loop(0, GROUPS, group, 0)

    pltpu.sync_copy(out_v, out_hbm.at[pl.ds(base * OUT_D, IPW * OUT_D)])


@functools.cache
def _build():
    mesh = plsc.VectorSubcoreMesh(
        core_axis_name="c", subcore_axis_name="s",
        num_cores=NC, num_subcores=NS)
    return pl.kernel(
        _sc_body,
        out_type=jax.ShapeDtypeStruct((K * OUT_D,), jnp.float32),
        mesh=mesh,
        compiler_params=pltpu.CompilerParams(needs_layout_passes=False),
        scratch_types=[
            pltpu.VMEM((N_EMB_WORDS,), jnp.float32),
            pltpu.VMEM((IPW,), jnp.int32),
            pltpu.VMEM((IPW,), jnp.int32),
            pltpu.VMEM((IPW,), jnp.int32),
            pltpu.VMEM((IPW * OUT_D,), jnp.float32),
        ],
    )


def kernel(type_emb, type_ids, hex1, hex2):
    emb = type_emb.reshape(-1).astype(jnp.float32)
    tid = type_ids.astype(jnp.int32)
    h1 = hex1.astype(jnp.int32)
    h2 = hex2.astype(jnp.int32)
    out = _build()(emb, tid, h1, h2)
    return out.reshape(K, OUT_D)


# speedup vs baseline: 1.8888x; 1.0162x over previous
import functools

import jax
import jax.numpy as jnp
from jax import lax
from jax.experimental import pallas as pl
from jax.experimental.pallas import tpu as pltpu
from jax.experimental.pallas import tpu_sc as plsc

WIDTH_FULL = 17
WIDTH_PLAYABLE = 15
HEIGHT = 11
TYPE_EMB_DIM = 8
K = 16384
OUT_D = 14
LANES = 16
NC, NS = 2, 16
NW = NC * NS
IPW = K // NW
GROUPS = IPW // LANES
N_EMB_WORDS = 4 * TYPE_EMB_DIM


def _sc_body(emb_hbm, tid_hbm, h1_hbm, h2_hbm, out_hbm,
             emb_v, tid_v, h1_v, h2_v, out_v, sem_in):
    wid = lax.axis_index("s") * NC + lax.axis_index("c")
    base = wid * IPW

    cps = [
        pltpu.async_copy(emb_hbm, emb_v, sem_in),
        pltpu.async_copy(tid_hbm.at[pl.ds(base, IPW)], tid_v, sem_in),
        pltpu.async_copy(h1_hbm.at[pl.ds(base, IPW)], h1_v, sem_in),
        pltpu.async_copy(h2_hbm.at[pl.ds(base, IPW)], h2_v, sem_in),
    ]
    for cp in cps:
        cp.wait()

    lane = lax.iota(jnp.int32, LANES)

    @plsc.parallel_loop(0, GROUPS)
    def group(g):
        off = g * LANES
        tid = tid_v[pl.ds(off, LANES)]
        h1 = h1_v[pl.ds(off, LANES)]
        h2 = h2_v[pl.ds(off, LANES)]
        row = (off + lane) * OUT_D

        tid8 = tid * TYPE_EMB_DIM
        for j in range(TYPE_EMB_DIM):
            ej = plsc.load_gather(emb_v, [tid8 + j])
            plsc.store_scatter(out_v, [row + j], ej)

        for h, jb in ((h1, TYPE_EMB_DIM), (h2, TYPE_EMB_DIM + 3)):
            y = lax.div(h, WIDTH_FULL)
            x = h - y * WIDTH_FULL
            valid = h >= 0
            xf = jnp.minimum(x, WIDTH_PLAYABLE - 1).astype(jnp.float32)
            yf = jnp.minimum(y, HEIGHT - 1).astype(jnp.float32)
            xf = jnp.where(valid, xf * (1.0 / (WIDTH_PLAYABLE - 1)), 0.0)
            yf = jnp.where(valid, yf * (1.0 / (HEIGHT - 1)), 0.0)
            one = jnp.where(valid, 1.0, 0.0)
            plsc.store_scatter(out_v, [row + jb], xf)
            plsc.store_scatter(out_v, [row + jb + 1], yf)
            plsc.store_scatter(out_v, [row + jb + 2], one)

    pltpu.sync_copy(out_v, out_hbm.at[pl.ds(base * OUT_D, IPW * OUT_D)])


@functools.cache
def _build():
    mesh = plsc.VectorSubcoreMesh(
        core_axis_name="c", subcore_axis_name="s",
        num_cores=NC, num_subcores=NS)
    return pl.kernel(
        _sc_body,
        out_type=jax.ShapeDtypeStruct((K * OUT_D,), jnp.float32),
        mesh=mesh,
        compiler_params=pltpu.CompilerParams(needs_layout_passes=False),
        scratch_types=[
            pltpu.VMEM((N_EMB_WORDS,), jnp.float32),
            pltpu.VMEM((IPW,), jnp.int32),
            pltpu.VMEM((IPW,), jnp.int32),
            pltpu.VMEM((IPW,), jnp.int32),
            pltpu.VMEM((IPW * OUT_D,), jnp.float32),
            pltpu.SemaphoreType.DMA,
        ],
    )


def kernel(type_emb, type_ids, hex1, hex2):
    emb = type_emb.reshape(-1).astype(jnp.float32)
    tid = type_ids.astype(jnp.int32)
    h1 = hex1.astype(jnp.int32)
    h2 = hex2.astype(jnp.int32)
    out = _build()(emb, tid, h1, h2)
    return out.reshape(K, OUT_D)


# speedup vs baseline: 2.3131x; 1.2246x over previous
import functools

import jax
import jax.numpy as jnp
from jax import lax
from jax.experimental import pallas as pl
from jax.experimental.pallas import tpu as pltpu
from jax.experimental.pallas import tpu_sc as plsc

WIDTH_FULL = 17
WIDTH_PLAYABLE = 15
HEIGHT = 11
TYPE_EMB_DIM = 8
K = 16384
OUT_D = 14
LANES = 16
NC, NS = 2, 16
NW = NC * NS
IPW = K // NW
GROUPS = IPW // LANES
N_EMB_WORDS = 4 * TYPE_EMB_DIM


def _sc_body(emb_hbm, tid_hbm, h1_hbm, h2_hbm, out_hbm,
             emb_v, tid_v, h1_v, h2_v, out_v, sem_in):
    wid = lax.axis_index("s") * NC + lax.axis_index("c")
    base = wid * IPW

    cps = [
        pltpu.async_copy(emb_hbm, emb_v, sem_in),
        pltpu.async_copy(tid_hbm.at[pl.ds(base, IPW)], tid_v, sem_in),
        pltpu.async_copy(h1_hbm.at[pl.ds(base, IPW)], h1_v, sem_in),
        pltpu.async_copy(h2_hbm.at[pl.ds(base, IPW)], h2_v, sem_in),
    ]
    for cp in cps:
        cp.wait()

    lane = lax.iota(jnp.int32, LANES)
    cols = [jnp.full((LANES,), j, dtype=jnp.int32) for j in range(OUT_D)]

    @plsc.parallel_loop(0, GROUPS)
    def group(g):
        off = g * LANES
        tid = tid_v[pl.ds(off, LANES)]
        h1 = h1_v[pl.ds(off, LANES)]
        h2 = h2_v[pl.ds(off, LANES)]
        row = off + lane

        tid8 = tid * TYPE_EMB_DIM
        for j in range(TYPE_EMB_DIM):
            ej = plsc.load_gather(emb_v, [tid8 + j])
            plsc.store_scatter(out_v, [row, cols[j]], ej)

        for h, jb in ((h1, TYPE_EMB_DIM), (h2, TYPE_EMB_DIM + 3)):
            y = lax.div(h, WIDTH_FULL)
            x = h - y * WIDTH_FULL
            valid = h >= 0
            xf = jnp.minimum(x, WIDTH_PLAYABLE - 1).astype(jnp.float32)
            yf = jnp.minimum(y, HEIGHT - 1).astype(jnp.float32)
            xf = jnp.where(valid, xf * (1.0 / (WIDTH_PLAYABLE - 1)), 0.0)
            yf = jnp.where(valid, yf * (1.0 / (HEIGHT - 1)), 0.0)
            one = jnp.where(valid, 1.0, 0.0)
            plsc.store_scatter(out_v, [row, cols[jb]], xf)
            plsc.store_scatter(out_v, [row, cols[jb + 1]], yf)
            plsc.store_scatter(out_v, [row, cols[jb + 2]], one)

    pltpu.sync_copy(out_v, out_hbm.at[pl.ds(base, IPW), :])


@functools.cache
def _build():
    mesh = plsc.VectorSubcoreMesh(
        core_axis_name="c", subcore_axis_name="s",
        num_cores=NC, num_subcores=NS)
    return pl.kernel(
        _sc_body,
        out_type=jax.ShapeDtypeStruct((K, OUT_D), jnp.float32),
        mesh=mesh,
        compiler_params=pltpu.CompilerParams(
            needs_layout_passes=False, use_tc_tiling_on_sc=True),
        scratch_types=[
            pltpu.VMEM((N_EMB_WORDS,), jnp.float32),
            pltpu.VMEM((IPW,), jnp.int32),
            pltpu.VMEM((IPW,), jnp.int32),
            pltpu.VMEM((IPW,), jnp.int32),
            pltpu.VMEM((IPW, OUT_D), jnp.float32),
            pltpu.SemaphoreType.DMA,
        ],
    )


def kernel(type_emb, type_ids, hex1, hex2):
    emb = type_emb.reshape(-1).astype(jnp.float32)
    tid = type_ids.astype(jnp.int32)
    h1 = hex1.astype(jnp.int32)
    h2 = hex2.astype(jnp.int32)
    return _build()(emb, tid, h1, h2)


# speedup vs baseline: 3.3435x; 1.4455x over previous
import functools

import jax
import jax.numpy as jnp
from jax import lax
from jax.experimental import pallas as pl
from jax.experimental.pallas import tpu as pltpu
from jax.experimental.pallas import tpu_sc as plsc

WIDTH_FULL = 17
WIDTH_PLAYABLE = 15
HEIGHT = 11
TYPE_EMB_DIM = 8
K = 16384
OUT_D = 14
LANES = 16
NC, NS = 2, 16
NW = NC * NS
IPW = K // NW
GROUPS = IPW // LANES
N_EMB_WORDS = 4 * TYPE_EMB_DIM


def _sc_body(emb_hbm, tid_hbm, h1_hbm, h2_hbm, out_hbm,
             emb_v, tid_v, h1_v, h2_v, col_v, sem_in):
    wid = lax.axis_index("s") * NC + lax.axis_index("c")
    base = wid * IPW

    cps = [
        pltpu.async_copy(emb_hbm, emb_v, sem_in),
        pltpu.async_copy(tid_hbm.at[pl.ds(base, IPW)], tid_v, sem_in),
        pltpu.async_copy(h1_hbm.at[pl.ds(base, IPW)], h1_v, sem_in),
        pltpu.async_copy(h2_hbm.at[pl.ds(base, IPW)], h2_v, sem_in),
    ]
    for cp in cps:
        cp.wait()

    @plsc.parallel_loop(0, GROUPS)
    def group(g):
        off = g * LANES
        tid = tid_v[pl.ds(off, LANES)]
        h1 = h1_v[pl.ds(off, LANES)]
        h2 = h2_v[pl.ds(off, LANES)]

        tid8 = tid * TYPE_EMB_DIM
        for j in range(TYPE_EMB_DIM):
            col_v[j, pl.ds(off, LANES)] = plsc.load_gather(emb_v, [tid8 + j])

        for h, jb in ((h1, TYPE_EMB_DIM), (h2, TYPE_EMB_DIM + 3)):
            y = lax.div(h, WIDTH_FULL)
            x = h - y * WIDTH_FULL
            valid = h >= 0
            xf = jnp.minimum(x, WIDTH_PLAYABLE - 1).astype(jnp.float32)
            yf = jnp.minimum(y, HEIGHT - 1).astype(jnp.float32)
            col_v[jb, pl.ds(off, LANES)] = jnp.where(
                valid, xf * (1.0 / (WIDTH_PLAYABLE - 1)), 0.0)
            col_v[jb + 1, pl.ds(off, LANES)] = jnp.where(
                valid, yf * (1.0 / (HEIGHT - 1)), 0.0)
            col_v[jb + 2, pl.ds(off, LANES)] = jnp.where(valid, 1.0, 0.0)

    pltpu.sync_copy(col_v, out_hbm.at[:, pl.ds(base, IPW)])


@functools.cache
def _build():
    mesh = plsc.VectorSubcoreMesh(
        core_axis_name="c", subcore_axis_name="s",
        num_cores=NC, num_subcores=NS)
    return pl.kernel(
        _sc_body,
        out_type=jax.ShapeDtypeStruct((OUT_D, K), jnp.float32),
        mesh=mesh,
        compiler_params=pltpu.CompilerParams(
            needs_layout_passes=False, use_tc_tiling_on_sc=True),
        scratch_types=[
            pltpu.VMEM((N_EMB_WORDS,), jnp.float32),
            pltpu.VMEM((IPW,), jnp.int32),
            pltpu.VMEM((IPW,), jnp.int32),
            pltpu.VMEM((IPW,), jnp.int32),
            pltpu.VMEM((OUT_D, IPW), jnp.float32),
            pltpu.SemaphoreType.DMA,
        ],
    )


def kernel(type_emb, type_ids, hex1, hex2):
    emb = type_emb.reshape(-1).astype(jnp.float32)
    tid = type_ids.astype(jnp.int32)
    h1 = hex1.astype(jnp.int32)
    h2 = hex2.astype(jnp.int32)
    return _build()(emb, tid, h1, h2).T
